# baseline (speedup 1.0000x reference)
import jax
import jax.numpy as jnp
from jax import lax
from jax.experimental import pallas as pl
from jax.experimental.pallas import tpu as pltpu
from jax.experimental.pallas import tpu_sc as plsc

B_ROWS = 4096
SEQ = 200
DIMS = 64

NC = 2
NS = 16
NW = NC * NS

TOTAL = B_ROWS * SEQ
PER_W = TOTAL // NW
CHUNK = 128
N_CHUNKS = PER_W // CHUNK
JBLK = B_ROWS // CHUNK

NBUF = 4
BATCH = 4
N_BATCH = N_CHUNKS // BATCH
IRING = 2 * NBUF


def _tj(gc):
    return gc // JBLK, gc % JBLK


def _body(x_hbm, table_hbm, out_hbm, idxr, rows_v, tbuf, isems, gsems, osems):
    wid = lax.axis_index("s") * NC + lax.axis_index("c")
    iota16 = lax.iota(jnp.int32, 16)
    gbase = wid * N_CHUNKS

    def stage_idx(g, slot):
        t, j = _tj(gbase + g)
        pltpu.async_copy(
            x_hbm.at[t, pl.ds(j * CHUNK, CHUNK)], idxr.at[slot],
            isems.at[slot],
        )

    def wait_idx(slot):
        pltpu.make_async_copy(
            x_hbm.at[0, pl.ds(0, CHUNK)], idxr.at[slot], isems.at[slot]
        ).wait()

    def start_gather(g, slot, rslot):
        pltpu.async_copy(
            table_hbm.at[idxr.at[slot]], rows_v.at[rslot], gsems.at[rslot]
        )

    for q in range(IRING):
        stage_idx(q, q)
    for b in range(NBUF):
        wait_idx(b)
        start_gather(b, b, b)

    @pl.loop(0, N_BATCH // 2)
    def _(ko):
        for p in range(2):
            k = ko * 2 + p
            t, j0 = _tj(gbase + k * BATCH)

            @pl.when(ko >= 1)
            def _():
                pltpu.make_async_copy(
                    tbuf.at[p], out_hbm.at[0, :, pl.ds(0, BATCH)],
                    osems.at[p],
                ).wait()

            for jj in range(BATCH):
                g = k * BATCH + jj
                islot = p * 4 + jj
                islot_next = (1 - p) * 4 + jj
                jjv = jnp.full((16,), jj, jnp.int32)

                pltpu.make_async_copy(
                    table_hbm.at[idxr.at[islot]], rows_v.at[jj], gsems.at[jj]
                ).wait()

                @pl.loop(0, 8)
                def _(l):
                    ridx = iota16 + l * 16

                    @pl.loop(0, 2)
                    def _(d4):
                        for d2 in range(8):
                            rot = lax.bitwise_and(iota16 + (d4 * 8 + d2), 15)
                            i0r = lax.shift_right_logical(rot, 3)
                            i1r = (
                                lax.shift_left(lax.bitwise_and(rot, 7), 7)
                                + ridx
                            )
                            for c0 in range(0, DIMS, 16):
                                v = plsc.load_gather(
                                    rows_v.at[jj], [ridx, rot + c0]
                                )
                                plsc.store_scatter(
                                    tbuf.at[p],
                                    [i0r + (c0 // 8), jjv, i1r],
                                    v,
                                )

                @pl.when(g + IRING < N_CHUNKS)
                def _():
                    stage_idx(g + IRING, islot)

                @pl.when(g + NBUF < N_CHUNKS)
                def _():
                    wait_idx(islot_next)
                    start_gather(g + NBUF, islot_next, jj)

            pltpu.async_copy(
                tbuf.at[p], out_hbm.at[t, :, pl.ds(j0, BATCH)],
                osems.at[p],
            )

    for p in range(2):
        pltpu.make_async_copy(
            tbuf.at[p], out_hbm.at[0, :, pl.ds(0, BATCH)], osems.at[p]
        ).wait()


_lookup = pl.kernel(
    _body,
    out_type=jax.ShapeDtypeStruct((SEQ, DIMS // 8, JBLK, 8 * CHUNK), jnp.float32),
    mesh=plsc.VectorSubcoreMesh(core_axis_name="c", subcore_axis_name="s"),
    scratch_types=[
        pltpu.VMEM((IRING, CHUNK), jnp.int32),
        pltpu.VMEM((NBUF, CHUNK, DIMS), jnp.float32),
        pltpu.VMEM((2, DIMS // 8, BATCH, 8 * CHUNK), jnp.float32),
        pltpu.SemaphoreType.DMA((IRING,)),
        pltpu.SemaphoreType.DMA((NBUF,)),
        pltpu.SemaphoreType.DMA((2,)),
    ],
    compiler_params=pltpu.CompilerParams(
        use_tc_tiling_on_sc=False, needs_layout_passes=False
    ),
)


@jax.jit
def kernel(x, table):
    out4 = _lookup(x.astype(jnp.int32).T, table)
    r = out4.reshape(SEQ, DIMS // 8, JBLK, 8, CHUNK)
    return r.transpose(2, 4, 0, 1, 3).reshape(B_ROWS, SEQ, DIMS)
